# baseline (speedup 1.0000x reference)
import functools

import jax
import jax.numpy as jnp
from jax import lax
from jax.experimental import pallas as pl
from jax.experimental.pallas import tpu as pltpu
from jax.experimental.pallas import tpu_sc as plsc

_B = 256
_S = 512
_E = 128
_NSEG = 3
_EPS = 1e-6

_NC = 2
_NS = 16
_NW = _NC * _NS
_TOK = _B * _S
_PERW = _TOK // _NW
_C = 128
_NCHUNK = _PERW // _C
_L = 16
_NV = _E // _L


def _rsqrt16(x):
    i = lax.bitcast_convert_type(x, jnp.int32)
    i = jnp.int32(0x5F3759DF) - (i >> 1)
    y = lax.bitcast_convert_type(i, jnp.float32)
    half = x * 0.5
    for _ in range(2):
        y = y * (1.5 - half * y * y)
    return y


def _sc_body(wtab, pstab, pttab, widx, psidx, ptidx, gam, bet, out,
             widx_v, psidx_v, ptidx_v, gam_v, bet_v, pt_v,
             buf0, buf1, buf2, sem0, sem1, sem2):
    wid = lax.axis_index("s") * _NC + lax.axis_index("c")

    pltpu.sync_copy(widx.at[wid], widx_v)
    pltpu.sync_copy(psidx.at[wid], psidx_v)
    pltpu.sync_copy(ptidx.at[wid], ptidx_v)
    pltpu.sync_copy(gam, gam_v)
    pltpu.sync_copy(bet, bet_v)
    pltpu.sync_copy(pttab, pt_v)

    base = wid * _PERW
    bufs = ((buf0, sem0), (buf1, sem1), (buf2, sem2))

    def fire_word(k, b):
        buf, sem = bufs[b]
        pltpu.async_copy(wtab.at[widx_v.at[k]], buf, sem)

    def drain_word(k, b):
        buf, sem = bufs[b]
        pltpu.make_async_copy(wtab.at[widx_v.at[k]], buf, sem).wait()

    def fire_adds(k, b):
        buf, sem = bufs[b]
        pltpu.async_copy(pstab.at[psidx_v.at[k]], buf, sem, add=True)

    def drain_adds(k, b):
        buf, sem = bufs[b]
        pltpu.make_async_copy(pstab.at[psidx_v.at[k]], buf, sem).wait()

    def fire_out(k, b):
        buf, sem = bufs[b]
        pltpu.async_copy(buf, out.at[pl.ds(base + k * _C, _C)], sem)

    def drain_out(k, b):
        buf, sem = bufs[b]
        pltpu.make_async_copy(
            buf, out.at[pl.ds(base + k * _C, _C)], sem).wait()

    lanes = lax.iota(jnp.int32, _L)
    gdn = lax.GatherDimensionNumbers(
        offset_dims=(), collapsed_slice_dims=(0,), start_index_map=(0,))

    def shuffle(x, idx):
        return lax.gather(
            x, idx[:, None], gdn, slice_sizes=(1,),
            mode=lax.GatherScatterMode.PROMISE_IN_BOUNDS)

    def hsum(x):
        for k in (1, 2, 4, 8):
            x = x + shuffle(x, lanes ^ k)
        return x

    def compute_chunk(k, b):
        buf, _ = bufs[b]

        @plsc.parallel_loop(0, _C // _L, unroll=1)
        def grp_body(g):
            t0 = g * _L
            ids16 = ptidx_v[k, pl.ds(t0, _L)] * _E
            for j in range(_L):
                pid = ids16[j]
                rw = buf.at[t0 + j]
                xs = []
                s1 = jnp.zeros((_L,), jnp.float32)
                s2 = jnp.zeros((_L,), jnp.float32)
                for v in range(_NV):
                    x = (rw[pl.ds(v * _L, _L)]
                         + pt_v[pl.ds(pid + v * _L, _L)])
                    xs.append(x)
                    s1 = s1 + x
                    s2 = s2 + x * x
                meanv = hsum(s1) * (1.0 / _E)
                varv = hsum(s2) * (1.0 / _E) - meanv * meanv
                inv = _rsqrt16(varv + _EPS)
                for v in range(_NV):
                    sl = pl.ds(v * _L, _L)
                    y = (xs[v] - meanv) * inv * gam_v[sl] + bet_v[sl]
                    rw[sl] = y

        fire_out(k, b)

    fire_word(0, 0)
    fire_word(1, 1)
    drain_word(0, 0)
    fire_adds(0, 0)

    @pl.loop(0, _NCHUNK, step=3)
    def chunk_triple(c):
        for b in (0, 1, 2):
            k = c + b

            @pl.when(jnp.logical_and(k >= 1, k + 2 < _NCHUNK))
            def _():
                drain_out(k - 1, (b + 2) % 3)

            @pl.when(k + 2 < _NCHUNK)
            def _():
                fire_word(k + 2, (b + 2) % 3)

            @pl.when(k + 1 < _NCHUNK)
            def _():
                drain_word(k + 1, (b + 1) % 3)
                fire_adds(k + 1, (b + 1) % 3)

            @pl.when(k < _NCHUNK)
            def _():
                drain_adds(k, b)
                compute_chunk(k, b)

    drain_out(_NCHUNK - 3, (_NCHUNK - 3) % 3)
    drain_out(_NCHUNK - 2, (_NCHUNK - 2) % 3)
    drain_out(_NCHUNK - 1, (_NCHUNK - 1) % 3)


def kernel(src, postag_ids, seg, word_table, pos_table, seg_table,
           postag_table, gamma, beta):
    ps_table = (pos_table[:, None, :] + seg_table[None, :, :]).reshape(
        _S * _NSEG, _E)

    src_i = src.astype(jnp.int32).reshape(_NW, _NCHUNK, _C)
    pos_ids = jnp.arange(_S, dtype=jnp.int32)
    ps_idx = (pos_ids[None, :] * _NSEG + seg.astype(jnp.int32)).reshape(
        _NW, _NCHUNK, _C)
    pt_idx = postag_ids.astype(jnp.int32).reshape(_NW, _NCHUNK, _C)

    mesh = plsc.VectorSubcoreMesh(core_axis_name="c", subcore_axis_name="s")
    run = functools.partial(
        pl.kernel,
        mesh=mesh,
        out_type=jax.ShapeDtypeStruct((_TOK, _E), jnp.float32),
        scratch_types=[
            pltpu.VMEM((_NCHUNK, _C), jnp.int32),
            pltpu.VMEM((_NCHUNK, _C), jnp.int32),
            pltpu.VMEM((_NCHUNK, _C), jnp.int32),
            pltpu.VMEM((_E,), jnp.float32),
            pltpu.VMEM((_E,), jnp.float32),
            pltpu.VMEM((50 * _E,), jnp.float32),
            pltpu.VMEM((_C, _E), jnp.float32),
            pltpu.VMEM((_C, _E), jnp.float32),
            pltpu.VMEM((_C, _E), jnp.float32),
            pltpu.SemaphoreType.DMA,
            pltpu.SemaphoreType.DMA,
            pltpu.SemaphoreType.DMA,
        ],
    )(_sc_body)
    out = run(word_table, ps_table, postag_table.reshape(-1), src_i, ps_idx,
              pt_idx, gamma, beta)
    return out.reshape(_B, _S, _E)


# speedup vs baseline: 3.3048x; 3.3048x over previous
import functools

import jax
import jax.numpy as jnp
from jax import lax
from jax.experimental import pallas as pl
from jax.experimental.pallas import tpu as pltpu
from jax.experimental.pallas import tpu_sc as plsc

_B = 256
_S = 512
_E = 128
_NSEG = 3
_EPS = 1e-6

_NC = 2
_NS = 16
_NW = _NC * _NS
_TOK = _B * _S
_PERW = _TOK // _NW
_C = 128
_NCHUNK = _PERW // _C
_L = 16
_NV = _E // _L


def _rsqrt16(x):
    i = lax.bitcast_convert_type(x, jnp.int32)
    i = jnp.int32(0x5F3759DF) - (i >> 1)
    y = lax.bitcast_convert_type(i, jnp.float32)
    half = x * 0.5
    for _ in range(2):
        y = y * (1.5 - half * y * y)
    return y


def _sc_body(wtab, pstab, pttab, widx, psidx, ptidx, gam, bet, out,
             widx_v, psidx_v, ptidx_v, gam_v, bet_v,
             buf0, buf1, buf2, sem0, sem1, sem2):
    wid = lax.axis_index("s") * _NC + lax.axis_index("c")

    pltpu.sync_copy(widx.at[wid], widx_v)
    pltpu.sync_copy(psidx.at[wid], psidx_v)
    pltpu.sync_copy(ptidx.at[wid], ptidx_v)
    pltpu.sync_copy(gam, gam_v)
    pltpu.sync_copy(bet, bet_v)

    base = wid * _PERW
    bufs = ((buf0, sem0), (buf1, sem1), (buf2, sem2))

    def fire_word(k, b):
        buf, sem = bufs[b]
        pltpu.async_copy(wtab.at[widx_v.at[k]], buf, sem)

    def drain_word(k, b):
        buf, sem = bufs[b]
        pltpu.make_async_copy(wtab.at[widx_v.at[k]], buf, sem).wait()

    def fire_adds(k, b):
        buf, sem = bufs[b]
        pltpu.async_copy(pstab.at[psidx_v.at[k]], buf, sem, add=True)
        pltpu.async_copy(pttab.at[ptidx_v.at[k]], buf, sem, add=True)

    def drain_adds(k, b):
        buf, sem = bufs[b]
        pltpu.make_async_copy(pstab.at[psidx_v.at[k]], buf, sem).wait()
        pltpu.make_async_copy(pttab.at[ptidx_v.at[k]], buf, sem).wait()

    def fire_out(k, b):
        buf, sem = bufs[b]
        pltpu.async_copy(buf, out.at[pl.ds(base + k * _C, _C)], sem)

    def drain_out(k, b):
        buf, sem = bufs[b]
        pltpu.make_async_copy(
            buf, out.at[pl.ds(base + k * _C, _C)], sem).wait()

    lanes = lax.iota(jnp.int32, _L)
    gdn = lax.GatherDimensionNumbers(
        offset_dims=(), collapsed_slice_dims=(0,), start_index_map=(0,))

    def shuffle(x, idx):
        return lax.gather(
            x, idx[:, None], gdn, slice_sizes=(1,),
            mode=lax.GatherScatterMode.PROMISE_IN_BOUNDS)

    def hsum(x):
        for k in (1, 2, 4, 8):
            x = x + shuffle(x, lanes ^ k)
        return x

    def compute_chunk(k, b):
        buf, _ = bufs[b]

        @plsc.parallel_loop(0, _C, unroll=2)
        def tok_body(t):
            rw = buf.at[t]
            xs = []
            s1 = jnp.zeros((_L,), jnp.float32)
            s2 = jnp.zeros((_L,), jnp.float32)
            for v in range(_NV):
                x = rw[pl.ds(v * _L, _L)]
                xs.append(x)
                s1 = s1 + x
                s2 = s2 + x * x
            meanv = hsum(s1) * (1.0 / _E)
            varv = hsum(s2) * (1.0 / _E) - meanv * meanv
            inv = _rsqrt16(varv + _EPS)
            for v in range(_NV):
                sl = pl.ds(v * _L, _L)
                y = (xs[v] - meanv) * inv * gam_v[sl] + bet_v[sl]
                rw[sl] = y

        fire_out(k, b)

    fire_word(0, 0)
    fire_word(1, 1)
    drain_word(0, 0)
    fire_adds(0, 0)

    @pl.loop(0, _NCHUNK, step=3)
    def chunk_triple(c):
        for b in (0, 1, 2):
            k = c + b

            @pl.when(jnp.logical_and(k >= 1, k + 2 < _NCHUNK))
            def _():
                drain_out(k - 1, (b + 2) % 3)

            @pl.when(k + 2 < _NCHUNK)
            def _():
                fire_word(k + 2, (b + 2) % 3)

            @pl.when(k + 1 < _NCHUNK)
            def _():
                drain_word(k + 1, (b + 1) % 3)
                fire_adds(k + 1, (b + 1) % 3)

            @pl.when(k < _NCHUNK)
            def _():
                drain_adds(k, b)
                compute_chunk(k, b)

    drain_out(_NCHUNK - 3, (_NCHUNK - 3) % 3)
    drain_out(_NCHUNK - 2, (_NCHUNK - 2) % 3)
    drain_out(_NCHUNK - 1, (_NCHUNK - 1) % 3)


def kernel(src, postag_ids, seg, word_table, pos_table, seg_table,
           postag_table, gamma, beta):
    ps_table = (pos_table[:, None, :] + seg_table[None, :, :]).reshape(
        _S * _NSEG, _E)

    src_i = src.astype(jnp.int32).reshape(_NW, _NCHUNK, _C)
    pos_ids = jnp.arange(_S, dtype=jnp.int32)
    ps_idx = (pos_ids[None, :] * _NSEG + seg.astype(jnp.int32)).reshape(
        _NW, _NCHUNK, _C)
    pt_idx = postag_ids.astype(jnp.int32).reshape(_NW, _NCHUNK, _C)

    mesh = plsc.VectorSubcoreMesh(core_axis_name="c", subcore_axis_name="s")
    run = functools.partial(
        pl.kernel,
        mesh=mesh,
        out_type=jax.ShapeDtypeStruct((_TOK, _E), jnp.float32),
        scratch_types=[
            pltpu.VMEM((_NCHUNK, _C), jnp.int32),
            pltpu.VMEM((_NCHUNK, _C), jnp.int32),
            pltpu.VMEM((_NCHUNK, _C), jnp.int32),
            pltpu.VMEM((_E,), jnp.float32),
            pltpu.VMEM((_E,), jnp.float32),
            pltpu.VMEM((_C, _E), jnp.float32),
            pltpu.VMEM((_C, _E), jnp.float32),
            pltpu.VMEM((_C, _E), jnp.float32),
            pltpu.SemaphoreType.DMA,
            pltpu.SemaphoreType.DMA,
            pltpu.SemaphoreType.DMA,
        ],
    )(_sc_body)
    out = run(word_table, ps_table, postag_table, src_i, ps_idx,
              pt_idx, gamma, beta)
    return out.reshape(_B, _S, _E)


# speedup vs baseline: 7.4722x; 2.2610x over previous
import functools

import jax
import jax.numpy as jnp
from jax import lax
from jax.experimental import pallas as pl
from jax.experimental.pallas import tpu as pltpu
from jax.experimental.pallas import tpu_sc as plsc

_B = 256
_S = 512
_E = 128
_NSEG = 3
_EPS = 1e-6

_NC = 2
_NS = 16
_NW = _NC * _NS
_TOK = _B * _S
_PERW = _TOK // _NW
_C = 128
_NCHUNK = _PERW // _C
_L = 16
_NV = _E // _L


def _rsqrt16(x):
    i = lax.bitcast_convert_type(x, jnp.int32)
    i = jnp.int32(0x5F3759DF) - (i >> 1)
    y = lax.bitcast_convert_type(i, jnp.float32)
    half = x * 0.5
    for _ in range(2):
        y = y * (1.5 - half * y * y)
    return y


def _sc_body(wtab, pstab, pttab, widx, psidx, ptidx, gam, bet, out,
             widx_v, psidx_v, ptidx_v, gam_v, bet_v,
             buf0, buf1, buf2, ps_s, pt_s, sem0, sem1, sem2):
    wid = lax.axis_index("s") * _NC + lax.axis_index("c")

    pltpu.sync_copy(widx.at[wid], widx_v)
    pltpu.sync_copy(psidx.at[wid], psidx_v)
    pltpu.sync_copy(ptidx.at[wid], ptidx_v)
    pltpu.sync_copy(gam, gam_v)
    pltpu.sync_copy(bet, bet_v)

    @pl.when(lax.axis_index("s") == 0)
    def _():
        pltpu.sync_copy(pstab, ps_s)
        pltpu.sync_copy(pttab, pt_s)

    plsc.subcore_barrier()

    base = wid * _PERW
    bufs = ((buf0, sem0), (buf1, sem1), (buf2, sem2))

    def fire_word(k, b):
        buf, sem = bufs[b]
        pltpu.async_copy(wtab.at[widx_v.at[k]], buf, sem)

    def drain_word(k, b):
        buf, sem = bufs[b]
        pltpu.make_async_copy(wtab.at[widx_v.at[k]], buf, sem).wait()

    def fire_adds(k, b):
        buf, sem = bufs[b]
        pltpu.async_copy(ps_s.at[psidx_v.at[k]], buf, sem, add=True)
        pltpu.async_copy(pt_s.at[ptidx_v.at[k]], buf, sem, add=True)

    def drain_adds(k, b):
        buf, sem = bufs[b]
        pltpu.make_async_copy(ps_s.at[psidx_v.at[k]], buf, sem).wait()
        pltpu.make_async_copy(pt_s.at[ptidx_v.at[k]], buf, sem).wait()

    def fire_out(k, b):
        buf, sem = bufs[b]
        pltpu.async_copy(buf, out.at[pl.ds(base + k * _C, _C)], sem)

    def drain_out(k, b):
        buf, sem = bufs[b]
        pltpu.make_async_copy(
            buf, out.at[pl.ds(base + k * _C, _C)], sem).wait()

    lanes = lax.iota(jnp.int32, _L)
    gdn = lax.GatherDimensionNumbers(
        offset_dims=(), collapsed_slice_dims=(0,), start_index_map=(0,))

    def shuffle(x, idx):
        return lax.gather(
            x, idx[:, None], gdn, slice_sizes=(1,),
            mode=lax.GatherScatterMode.PROMISE_IN_BOUNDS)

    def hsum(x):
        for k in (1, 2, 4, 8):
            x = x + shuffle(x, lanes ^ k)
        return x

    def compute_chunk(k, b):
        buf, _ = bufs[b]

        @plsc.parallel_loop(0, _C, unroll=2)
        def tok_body(t):
            rw = buf.at[t]
            xs = []
            s1 = jnp.zeros((_L,), jnp.float32)
            s2 = jnp.zeros((_L,), jnp.float32)
            for v in range(_NV):
                x = rw[pl.ds(v * _L, _L)]
                xs.append(x)
                s1 = s1 + x
                s2 = s2 + x * x
            meanv = hsum(s1) * (1.0 / _E)
            varv = hsum(s2) * (1.0 / _E) - meanv * meanv
            inv = _rsqrt16(varv + _EPS)
            for v in range(_NV):
                sl = pl.ds(v * _L, _L)
                y = (xs[v] - meanv) * inv * gam_v[sl] + bet_v[sl]
                rw[sl] = y

        fire_out(k, b)

    fire_word(0, 0)
    fire_word(1, 1)
    drain_word(0, 0)
    fire_adds(0, 0)

    @pl.loop(0, _NCHUNK, step=3)
    def chunk_triple(c):
        for b in (0, 1, 2):
            k = c + b

            @pl.when(jnp.logical_and(k >= 1, k + 2 < _NCHUNK))
            def _():
                drain_out(k - 1, (b + 2) % 3)

            @pl.when(k + 2 < _NCHUNK)
            def _():
                fire_word(k + 2, (b + 2) % 3)

            @pl.when(k + 1 < _NCHUNK)
            def _():
                drain_word(k + 1, (b + 1) % 3)
                fire_adds(k + 1, (b + 1) % 3)

            @pl.when(k < _NCHUNK)
            def _():
                drain_adds(k, b)
                compute_chunk(k, b)

    drain_out(_NCHUNK - 3, (_NCHUNK - 3) % 3)
    drain_out(_NCHUNK - 2, (_NCHUNK - 2) % 3)
    drain_out(_NCHUNK - 1, (_NCHUNK - 1) % 3)


def kernel(src, postag_ids, seg, word_table, pos_table, seg_table,
           postag_table, gamma, beta):
    ps_table = (pos_table[:, None, :] + seg_table[None, :, :]).reshape(
        _S * _NSEG, _E)

    src_i = src.astype(jnp.int32).reshape(_NW, _NCHUNK, _C)
    pos_ids = jnp.arange(_S, dtype=jnp.int32)
    ps_idx = (pos_ids[None, :] * _NSEG + seg.astype(jnp.int32)).reshape(
        _NW, _NCHUNK, _C)
    pt_idx = postag_ids.astype(jnp.int32).reshape(_NW, _NCHUNK, _C)

    mesh = plsc.VectorSubcoreMesh(core_axis_name="c", subcore_axis_name="s")
    run = functools.partial(
        pl.kernel,
        mesh=mesh,
        out_type=jax.ShapeDtypeStruct((_TOK, _E), jnp.float32),
        scratch_types=[
            pltpu.VMEM((_NCHUNK, _C), jnp.int32),
            pltpu.VMEM((_NCHUNK, _C), jnp.int32),
            pltpu.VMEM((_NCHUNK, _C), jnp.int32),
            pltpu.VMEM((_E,), jnp.float32),
            pltpu.VMEM((_E,), jnp.float32),
            pltpu.VMEM((_C, _E), jnp.float32),
            pltpu.VMEM((_C, _E), jnp.float32),
            pltpu.VMEM((_C, _E), jnp.float32),
            pltpu.VMEM_SHARED((_S * _NSEG, _E), jnp.float32),
            pltpu.VMEM_SHARED((50, _E), jnp.float32),
            pltpu.SemaphoreType.DMA,
            pltpu.SemaphoreType.DMA,
            pltpu.SemaphoreType.DMA,
        ],
    )(_sc_body)
    out = run(word_table, ps_table, postag_table, src_i, ps_idx,
              pt_idx, gamma, beta)
    return out.reshape(_B, _S, _E)
